# Initial kernel scaffold; baseline (speedup 1.0000x reference)
import jax
import jax.numpy as jnp
from jax.experimental import pallas as pl


def kernel(embedding):
    raise NotImplementedError("write your pallas kernel here")


# speedup vs baseline: 8.1359x; 8.1359x over previous
import functools

import jax
import jax.numpy as jnp
from jax import lax
from jax.experimental import pallas as pl
from jax.experimental.pallas import tpu as pltpu
from jax.experimental.pallas import tpu_sc as plsc

_SEQ = 2048
_UNITS = 32
_NC = 2
_NS = 16
_NW = _NC * _NS
_ROWS = _SEQ // _NW
_ROW_F = _SEQ * _UNITS
_WIN_D = _SEQ + _ROWS
_WIN_F = _WIN_D * _UNITS
_FIRE = 8


def _sc_body(emb_hbm, out_hbm, emb_v, win_v, *sems):
    cid = lax.axis_index("c")
    sid = lax.axis_index("s")
    wid = sid * _NC + cid

    pltpu.sync_copy(emb_hbm, emb_v)
    halves = [(emb_v[v, pl.ds(0, 16)], emb_v[v, pl.ds(16, 16)]) for v in range(5)]

    t1 = 62 + 64 * wid

    def fill_run(lo, hi, h):
        def body(ld, c):
            win_v[pl.ds(ld * _UNITS, 16)] = h[0]
            win_v[pl.ds(ld * _UNITS + 16, 16)] = h[1]
            return c
        lax.fori_loop(lo, hi, body, 0)

    fill_run(0, t1, halves[0])
    for k in range(3):
        base = (t1 + k) * _UNITS
        win_v[pl.ds(base, 16)] = halves[k + 1][0]
        win_v[pl.ds(base + 16, 16)] = halves[k + 1][1]
    fill_run(t1 + 3, _WIN_D, halves[4])

    row0 = wid * _ROWS
    for g in range(0, _ROWS, _FIRE):
        copies = []
        for r in range(g, g + _FIRE):
            src = win_v.at[pl.ds((_ROWS - 1 - r) * _UNITS, _ROW_F)]
            dst = out_hbm.at[pl.ds((row0 + r) * _ROW_F, _ROW_F)]
            copies.append(pltpu.async_copy(src, dst, sems[r - g]))
        for c in copies:
            c.wait()


_rel_pos_sc = functools.partial(
    pl.kernel,
    out_type=jax.ShapeDtypeStruct((_SEQ * _SEQ * _UNITS,), jnp.float32),
    mesh=plsc.VectorSubcoreMesh(core_axis_name="c", subcore_axis_name="s"),
    scratch_types=(
        [pltpu.VMEM((2 * 2 + 1, _UNITS), jnp.float32),
         pltpu.VMEM((_WIN_F,), jnp.float32)]
        + [pltpu.SemaphoreType.DMA] * _FIRE
    ),
)(_sc_body)


def kernel(embedding):
    flat = _rel_pos_sc(embedding)
    return flat.reshape(_SEQ, _SEQ, _UNITS)
